# Initial kernel scaffold; baseline (speedup 1.0000x reference)
import jax
import jax.numpy as jnp
from jax.experimental import pallas as pl


def kernel(x, edge_index, Wl1, Wr1, att1, b1, Wl2, Wr2, att2, b2):
    raise NotImplementedError("write your pallas kernel here")


# speedup vs baseline: 23.8195x; 23.8195x over previous
import functools

import jax
import jax.numpy as jnp
from jax import lax
from jax.experimental import pallas as pl
from jax.experimental.pallas import tpu as pltpu
from jax.experimental.pallas import tpu_sc as plsc

N = 10000
E = 320000
ET = E + N
D_IN = 128
HID = 64
HEADS = 2
D_OUT = 64

NC, NS = 2, 16
NW = NC * NS
CH = 128
RPT = -(-ET // (CH * NW))
R = RPT * NW
NP = 10240
NPT = NP // NS
HW = 64
V = HW // 16
W = 128
BN = 1000
BC = 80


def _quad_mm_body(x_ref, w0_ref, w1_ref, w2_ref, w3_ref,
                  o0_ref, o1_ref, o2_ref, o3_ref):
    xb = x_ref[...]
    o0_ref[...] = jnp.dot(xb, w0_ref[...], preferred_element_type=jnp.float32)
    o1_ref[...] = jnp.dot(xb, w1_ref[...], preferred_element_type=jnp.float32)
    o2_ref[...] = jnp.dot(xb, w2_ref[...], preferred_element_type=jnp.float32)
    o3_ref[...] = jnp.dot(xb, w3_ref[...], preferred_element_type=jnp.float32)


def _quad_matmul(x, ws):
    n, d = x.shape
    return pl.pallas_call(
        _quad_mm_body,
        grid=(n // BN,),
        in_specs=[pl.BlockSpec((BN, d), lambda i: (i, 0))]
        + [pl.BlockSpec((d, HW), lambda i: (0, 0))] * 4,
        out_specs=[pl.BlockSpec((BN, HW), lambda i: (i, 0))] * 4,
        out_shape=[jax.ShapeDtypeStruct((n, HW), jnp.float32)] * 4,
    )(x, *ws)


def _make_edge_kernel(n_phase):
    mesh = plsc.VectorSubcoreMesh(core_axis_name="c", subcore_axis_name="s")

    @functools.partial(
        pl.kernel,
        out_type=jax.ShapeDtypeStruct((n_phase * NC * NP, W), jnp.float32),
        mesh=mesh,
        scratch_types=[
            pltpu.VMEM((CH,), jnp.int32),
            pltpu.VMEM((CH,), jnp.int32),
            pltpu.VMEM((CH, HW), jnp.float32),
            pltpu.VMEM((CH, HW), jnp.float32),
            pltpu.VMEM((CH, W), jnp.float32),
            pltpu.VMEM((n_phase * V, 16), jnp.float32),
            pltpu.VMEM_SHARED((NP, W), jnp.float32),
            pltpu.SemaphoreType.DMA,
            pltpu.SemaphoreType.DMA,
        ],
        compiler_params=pltpu.CompilerParams(use_tc_tiling_on_sc=False),
    )
    def edge_kernel(*refs):
        tables = refs[:2 * n_phase]
        (srow_hbm, drow_hbm, att_hbm, out_hbm,
         idx_s, idx_d, xj_v, xi_v, acc_v, att_v, accum,
         sem1, sem2) = refs[2 * n_phase:]
        c = lax.axis_index("c")
        s = lax.axis_index("s")
        wid = s * NC + c
        zero16 = jnp.zeros((16,), jnp.float32)
        lane = lax.broadcasted_iota(jnp.int32, (16,), 0)
        ind0 = jnp.maximum(1 - lane, 0).astype(jnp.float32)
        nbase = s * NPT
        nfull = NPT // CH

        pltpu.sync_copy(att_hbm, att_v)

        def _zrow(i, carry):
            for k in range(W // 16):
                acc_v[i, pl.ds(16 * k, 16)] = zero16
            return carry

        for ph in range(n_phase):
            xl_hbm = tables[2 * ph]
            xr_hbm = tables[2 * ph + 1]
            att_r = [att_v[V * ph + k, :] for k in range(V)]
            lax.fori_loop(0, CH, _zrow, 0)
            for t in range(nfull):
                pltpu.sync_copy(acc_v, accum.at[pl.ds(nbase + t * CH, CH)])
            plsc.subcore_barrier()

            def _chunk(j, carry):
                row = wid * RPT + j
                pltpu.sync_copy(srow_hbm.at[row], idx_s)
                pltpu.sync_copy(drow_hbm.at[row], idx_d)
                d1 = pltpu.async_copy(xl_hbm.at[idx_s], xj_v, sem1)
                d2 = pltpu.async_copy(xr_hbm.at[idx_d], xi_v, sem2)
                d1.wait()
                d2.wait()
                ebase = row * CH

                def _edge(e, ecarry):
                    xj_r = [xj_v[e, pl.ds(16 * k, 16)] for k in range(V)]
                    xi_r = [xi_v[e, pl.ds(16 * k, 16)] for k in range(V)]
                    tot = zero16
                    for k in range(V):
                        sv = xi_r[k] + xj_r[k]
                        ev = jnp.maximum(sv, 0.2 * sv)
                        tot = tot + ev * att_r[k]
                    for step in (8, 4, 2, 1):
                        tot = tot + tot.at[lane ^ step].get(
                            mode="promise_in_bounds")
                    gidv = jnp.full((16,), ebase + e, jnp.int32)
                    validf = jnp.minimum(jnp.maximum(ET - gidv, 0),
                                         1).astype(jnp.float32)
                    pv = jnp.exp(tot) * validf
                    for k in range(V):
                        acc_v[e, pl.ds(16 * k, 16)] = xj_r[k] * pv
                    acc_v[e, pl.ds(HW, 16)] = pv * ind0
                    return ecarry

                lax.fori_loop(0, CH, _edge, 0)
                pltpu.sync_copy(acc_v, accum.at[idx_d], add=True)
                return carry

            lax.fori_loop(0, RPT, _chunk, 0)
            plsc.subcore_barrier()
            obase = (ph * NC + c) * NP + nbase
            for t in range(nfull):
                pltpu.sync_copy(accum.at[pl.ds(nbase + t * CH, CH)],
                                out_hbm.at[pl.ds(obase + t * CH, CH)])

    return edge_kernel


_edge_l1 = _make_edge_kernel(HEADS)
_edge_l2 = _make_edge_kernel(1)


def _combine1_body(p00_ref, p01_ref, p10_ref, p11_ref, b_ref, wl_ref, wr_ref,
                   hl_ref, hr_ref):
    acc0 = p00_ref[...] + p01_ref[...]
    acc1 = p10_ref[...] + p11_ref[...]
    h0 = acc0[:, :HID] / acc0[:, HID:HID + 1]
    h1 = acc1[:, :HID] / acc1[:, HID:HID + 1]
    h = jnp.concatenate([h0, h1], axis=1) + b_ref[...]
    h = jnp.where(h > 0, h, 0.01 * h)
    hl_ref[...] = jnp.dot(h, wl_ref[...], preferred_element_type=jnp.float32)
    hr_ref[...] = jnp.dot(h, wr_ref[...], preferred_element_type=jnp.float32)


def _combine1(parts, b, wl, wr):
    off = NP // BC

    def spec(q):
        return pl.BlockSpec((BC, W), lambda i, q=q: (i + q * off, 0))

    return pl.pallas_call(
        _combine1_body,
        grid=(N // BC,),
        in_specs=[
            spec(0), spec(1), spec(2), spec(3),
            pl.BlockSpec((1, HEADS * HID), lambda i: (0, 0)),
            pl.BlockSpec((HEADS * HID, D_OUT), lambda i: (0, 0)),
            pl.BlockSpec((HEADS * HID, D_OUT), lambda i: (0, 0)),
        ],
        out_specs=[pl.BlockSpec((BC, D_OUT), lambda i: (i, 0))] * 2,
        out_shape=[jax.ShapeDtypeStruct((N, D_OUT), jnp.float32)] * 2,
    )(parts, parts, parts, parts, b, wl, wr)


def _combine2_body(p0_ref, p1_ref, b_ref, o_ref):
    acc = p0_ref[...] + p1_ref[...]
    o_ref[...] = acc[:, :D_OUT] / acc[:, D_OUT:D_OUT + 1] + b_ref[...]


def _combine2(parts, b):
    off = NP // BC
    return pl.pallas_call(
        _combine2_body,
        grid=(N // BC,),
        in_specs=[
            pl.BlockSpec((BC, W), lambda i: (i, 0)),
            pl.BlockSpec((BC, W), lambda i: (i + off, 0)),
            pl.BlockSpec((1, D_OUT), lambda i: (0, 0)),
        ],
        out_specs=pl.BlockSpec((BC, D_OUT), lambda i: (i, 0)),
        out_shape=jax.ShapeDtypeStruct((N, D_OUT), jnp.float32),
    )(parts, parts, b)


def kernel(x, edge_index, Wl1, Wr1, att1, b1, Wl2, Wr2, att2, b2):
    xf = x.reshape(N, D_IN)
    loop = jnp.arange(N, dtype=edge_index.dtype)
    src = jnp.concatenate([edge_index[0], loop])
    dst = jnp.concatenate([edge_index[1], loop])
    pad = R * CH - ET
    src = jnp.pad(src, (0, pad)).reshape(R, CH)
    dst = jnp.pad(dst, (0, pad)).reshape(R, CH)

    xl0, xl1, xr0, xr1 = _quad_matmul(
        xf, (Wl1[:, :HID], Wl1[:, HID:], Wr1[:, :HID], Wr1[:, HID:]))
    parts1 = _edge_l1(xl0, xr0, xl1, xr1, src, dst,
                      att1.reshape(HEADS * V, 16))
    hl2, hr2 = _combine1(parts1, b1.reshape(1, -1), Wl2, Wr2)
    parts2 = _edge_l2(hl2, hr2, src, dst, att2.reshape(V, 16))
    out = _combine2(parts2, b2.reshape(1, -1))
    return out.reshape(1, N, D_OUT)


# speedup vs baseline: 33.2976x; 1.3979x over previous
import functools

import jax
import jax.numpy as jnp
from jax import lax
from jax.experimental import pallas as pl
from jax.experimental.pallas import tpu as pltpu
from jax.experimental.pallas import tpu_sc as plsc

N = 10000
E = 320000
ET = E + N
D_IN = 128
HID = 64
HEADS = 2
D_OUT = 64

NC, NS = 2, 16
NW = NC * NS
CH = 128
RPT = ((-(-ET // (CH * NW)) + 1) // 2) * 2
R = RPT * NW
NP = 10240
NACC = 10008
HW = 64
V = HW // 16
W = 80
BN = 1024
BC = 80


def _quad_mm_body(x_ref, w0_ref, w1_ref, w2_ref, w3_ref,
                  o0_ref, o1_ref, o2_ref, o3_ref):
    xb = x_ref[...]
    o0_ref[...] = jnp.dot(xb, w0_ref[...], preferred_element_type=jnp.float32)
    o1_ref[...] = jnp.dot(xb, w1_ref[...], preferred_element_type=jnp.float32)
    o2_ref[...] = jnp.dot(xb, w2_ref[...], preferred_element_type=jnp.float32)
    o3_ref[...] = jnp.dot(xb, w3_ref[...], preferred_element_type=jnp.float32)


def _quad_matmul(x, ws):
    n, d = x.shape
    return pl.pallas_call(
        _quad_mm_body,
        grid=(n // BN,),
        in_specs=[pl.BlockSpec((BN, d), lambda i: (i, 0))]
        + [pl.BlockSpec((d, HW), lambda i: (0, 0))] * 4,
        out_specs=[pl.BlockSpec((BN, HW), lambda i: (i, 0))] * 4,
        out_shape=[jax.ShapeDtypeStruct((n, HW), jnp.float32)] * 4,
    )(x, *ws)


def _make_edge_kernel(n_phase):
    mesh = plsc.VectorSubcoreMesh(core_axis_name="c", subcore_axis_name="s")
    scratch = (
        [pltpu.VMEM((CH,), jnp.int32)] * 2
        + [pltpu.VMEM((CH,), jnp.int32)] * 2
        + [pltpu.VMEM((CH,), jnp.int32)] * 2
        + [pltpu.VMEM((CH, HW), jnp.float32)] * 2
        + [pltpu.VMEM((CH, HW), jnp.float32)] * 2
        + [pltpu.VMEM((CH, W), jnp.float32)] * 2
        + [pltpu.VMEM((n_phase * V, 16), jnp.float32)]
        + [pltpu.VMEM_SHARED((NACC, W), jnp.float32)]
        + [pltpu.SemaphoreType.DMA] * 10
    )

    @functools.partial(
        pl.kernel,
        out_type=jax.ShapeDtypeStruct((n_phase * NC * NP, W), jnp.float32),
        mesh=mesh,
        scratch_types=scratch,
        compiler_params=pltpu.CompilerParams(use_tc_tiling_on_sc=False),
    )
    def edge_kernel(*refs):
        tables = refs[:2 * n_phase]
        (srow_hbm, drow_hbm, att_hbm, out_hbm) = refs[2 * n_phase:2 * n_phase + 4]
        sc = refs[2 * n_phase + 4:]
        idx_s = sc[0:2]
        idx_d = sc[2:4]
        idx_sc = sc[4:6]
        xj_v = sc[6:8]
        xi_v = sc[8:10]
        acc_v = sc[10:12]
        att_v = sc[12]
        accum = sc[13]
        semis = sc[14:16]
        semid = sc[16:18]
        semgs = sc[18:20]
        semgd = sc[20:22]
        semsc = sc[22:24]

        c = lax.axis_index("c")
        s = lax.axis_index("s")
        wid = s * NC + c
        rbase = wid * RPT
        zero16 = jnp.zeros((16,), jnp.float32)
        lane = lax.broadcasted_iota(jnp.int32, (16,), 0)
        ind0 = jnp.maximum(1 - lane, 0).astype(jnp.float32)
        nbase = 624 * s + 8 * jnp.maximum(s - 13, 0)

        pltpu.sync_copy(att_hbm, att_v)

        def _zrow(i, carry):
            for k in range(W // 16):
                acc_v[0][i, pl.ds(16 * k, 16)] = zero16
            return carry

        for ph in range(n_phase):
            xl_hbm = tables[2 * ph]
            xr_hbm = tables[2 * ph + 1]
            att_r = [att_v[V * ph + k, :] for k in range(V)]

            lax.fori_loop(0, CH, _zrow, 0)
            for t in range(4):
                pltpu.sync_copy(acc_v[0], accum.at[pl.ds(nbase + t * CH, CH)])
            pltpu.sync_copy(acc_v[0].at[pl.ds(0, 112)],
                            accum.at[pl.ds(nbase + 512, 112)])

            @pl.when(s >= 13)
            def _():
                pltpu.sync_copy(acc_v[0].at[pl.ds(0, 8)],
                                accum.at[pl.ds(nbase + 624, 8)])

            plsc.subcore_barrier()

            def _issue_idx(q, b):
                row = rbase + q
                pltpu.async_copy(srow_hbm.at[row], idx_s[b], semis[b])
                pltpu.async_copy(drow_hbm.at[row], idx_d[b], semid[b])

            def _wait_idx(b):
                pltpu.make_async_copy(srow_hbm.at[0], idx_s[b],
                                      semis[b]).wait()
                pltpu.make_async_copy(drow_hbm.at[0], idx_d[b],
                                      semid[b]).wait()

            def _issue_gather(b):
                pltpu.async_copy(xl_hbm.at[idx_s[b]], xj_v[b], semgs[b])
                pltpu.async_copy(xr_hbm.at[idx_d[b]], xi_v[b], semgd[b])

            def _wait_gather(b):
                pltpu.make_async_copy(xl_hbm.at[idx_s[b]], xj_v[b],
                                      semgs[b]).wait()
                pltpu.make_async_copy(xr_hbm.at[idx_d[b]], xi_v[b],
                                      semgd[b]).wait()

            def _wait_scatter(b):
                pltpu.make_async_copy(acc_v[b], accum.at[idx_sc[b]],
                                      semsc[b]).wait()

            def _compute_scatter(b):
                for k in range(CH // 16):
                    idx_sc[b][pl.ds(16 * k, 16)] = idx_d[b][pl.ds(16 * k, 16)]

                def _edge(e, ecarry):
                    xj_r = [xj_v[b][e, pl.ds(16 * k, 16)] for k in range(V)]
                    xi_r = [xi_v[b][e, pl.ds(16 * k, 16)] for k in range(V)]
                    tot = zero16
                    for k in range(V):
                        sv = xi_r[k] + xj_r[k]
                        ev = jnp.maximum(sv, 0.2 * sv)
                        tot = tot + ev * att_r[k]
                    for step in (8, 4, 2, 1):
                        tot = tot + tot.at[lane ^ step].get(
                            mode="promise_in_bounds")
                    pv = jnp.exp(tot)
                    for k in range(V):
                        acc_v[b][e, pl.ds(16 * k, 16)] = xj_r[k] * pv
                    acc_v[b][e, pl.ds(HW, 16)] = pv * ind0
                    return ecarry

                lax.fori_loop(0, CH, _edge, 0, unroll=2)
                pltpu.async_copy(acc_v[b], accum.at[idx_sc[b]], semsc[b],
                                 add=True)

            _issue_idx(0, 0)
            _wait_idx(0)
            _issue_gather(0)

            def _body(t, carry):
                q0 = 2 * t
                for i in range(2):
                    b = i

                    @pl.when(t >= 1)
                    def _():
                        _wait_scatter(b)

                    _issue_idx(q0 + i + 1, (i + 1) % 2)
                    _wait_idx((i + 1) % 2)
                    _issue_gather((i + 1) % 2)
                    _wait_gather(b)
                    _compute_scatter(b)
                return carry

            lax.fori_loop(0, (RPT - 2) // 2, _body, 0)
            for i in range(2):
                q = RPT - 2 + i
                _wait_scatter(i)
                if i == 0:
                    _issue_idx(q + 1, (i + 1) % 2)
                    _wait_idx((i + 1) % 2)
                    _issue_gather((i + 1) % 2)
                _wait_gather(i)
                _compute_scatter(i)
            for i in range(2):
                _wait_scatter(i)
            plsc.subcore_barrier()
            obase = (ph * NC + c) * NP + nbase
            for t in range(4):
                pltpu.sync_copy(accum.at[pl.ds(nbase + t * CH, CH)],
                                out_hbm.at[pl.ds(obase + t * CH, CH)])
            pltpu.sync_copy(accum.at[pl.ds(nbase + 512, 112)],
                            out_hbm.at[pl.ds(obase + 512, 112)])

            @pl.when(s >= 13)
            def _():
                pltpu.sync_copy(accum.at[pl.ds(nbase + 624, 8)],
                                out_hbm.at[pl.ds(obase + 624, 8)])

    return edge_kernel


_edge_l1 = _make_edge_kernel(HEADS)
_edge_l2 = _make_edge_kernel(1)


def _combine1_body(p00_ref, p01_ref, p10_ref, p11_ref, b_ref, wl_ref, wr_ref,
                   hl_ref, hr_ref):
    acc0 = p00_ref[...] + p01_ref[...]
    acc1 = p10_ref[...] + p11_ref[...]
    h0 = acc0[:, :HID] / acc0[:, HID:HID + 1]
    h1 = acc1[:, :HID] / acc1[:, HID:HID + 1]
    h = jnp.concatenate([h0, h1], axis=1) + b_ref[...]
    h = jnp.where(h > 0, h, 0.01 * h)
    hl_ref[...] = jnp.dot(h, wl_ref[...], preferred_element_type=jnp.float32)
    hr_ref[...] = jnp.dot(h, wr_ref[...], preferred_element_type=jnp.float32)


def _combine1(parts, b, wl, wr):
    off = NP // BC

    def spec(q):
        return pl.BlockSpec((BC, W), lambda i, q=q: (i + q * off, 0))

    return pl.pallas_call(
        _combine1_body,
        grid=(NP // BC,),
        in_specs=[
            spec(0), spec(1), spec(2), spec(3),
            pl.BlockSpec((1, HEADS * HID), lambda i: (0, 0)),
            pl.BlockSpec((HEADS * HID, D_OUT), lambda i: (0, 0)),
            pl.BlockSpec((HEADS * HID, D_OUT), lambda i: (0, 0)),
        ],
        out_specs=[pl.BlockSpec((BC, D_OUT), lambda i: (i, 0))] * 2,
        out_shape=[jax.ShapeDtypeStruct((NP, D_OUT), jnp.float32)] * 2,
    )(parts, parts, parts, parts, b, wl, wr)


def _combine2_body(p0_ref, p1_ref, b_ref, o_ref):
    acc = p0_ref[...] + p1_ref[...]
    o_ref[...] = acc[:, :D_OUT] / acc[:, D_OUT:D_OUT + 1] + b_ref[...]


def _combine2(parts, b):
    off = NP // BC
    return pl.pallas_call(
        _combine2_body,
        grid=(N // BC,),
        in_specs=[
            pl.BlockSpec((BC, W), lambda i: (i, 0)),
            pl.BlockSpec((BC, W), lambda i: (i + off, 0)),
            pl.BlockSpec((1, D_OUT), lambda i: (0, 0)),
        ],
        out_specs=pl.BlockSpec((BC, D_OUT), lambda i: (i, 0)),
        out_shape=jax.ShapeDtypeStruct((N, D_OUT), jnp.float32),
    )(parts, parts, b)


def kernel(x, edge_index, Wl1, Wr1, att1, b1, Wl2, Wr2, att2, b2):
    xf = jnp.pad(x.reshape(N, D_IN), ((0, NP - N), (0, 0)))
    loop = jnp.arange(N, dtype=edge_index.dtype)
    src = jnp.concatenate([edge_index[0], loop])
    dst = jnp.concatenate([edge_index[1], loop])
    pad = (R + 2) * CH - ET
    src = jnp.pad(src, (0, pad), constant_values=N).reshape(R + 2, CH)
    dst = jnp.pad(dst, (0, pad), constant_values=N).reshape(R + 2, CH)

    xl0, xl1, xr0, xr1 = _quad_matmul(
        xf, (Wl1[:, :HID], Wl1[:, HID:], Wr1[:, :HID], Wr1[:, HID:]))
    parts1 = _edge_l1(xl0, xr0, xl1, xr1, src, dst,
                      att1.reshape(HEADS * V, 16))
    hl2, hr2 = _combine1(parts1, b1.reshape(1, -1), Wl2, Wr2)
    parts2 = _edge_l2(hl2, hr2, src, dst, att2.reshape(V, 16))
    out = _combine2(parts2, b2.reshape(1, -1))
    return out.reshape(1, N, D_OUT)


# speedup vs baseline: 35.7660x; 1.0741x over previous
import functools

import jax
import jax.numpy as jnp
from jax import lax
from jax.experimental import pallas as pl
from jax.experimental.pallas import tpu as pltpu
from jax.experimental.pallas import tpu_sc as plsc

N = 10000
E = 320000
ET = E + N
D_IN = 128
HID = 64
HEADS = 2
D_OUT = 64

NC, NS = 2, 16
NW = NC * NS
CH = 128
RPT = ((-(-ET // (CH * NW)) + 1) // 2) * 2
R = RPT * NW
NP = 10240
NACC = 10008
HW = 64
V = HW // 16
W = 80
BN = 1024
BC = 80


def _quad_mm_body(x_ref, w0_ref, w1_ref, w2_ref, w3_ref,
                  o0_ref, o1_ref, o2_ref, o3_ref):
    xb = x_ref[...]
    o0_ref[...] = jnp.dot(xb, w0_ref[...], preferred_element_type=jnp.float32)
    o1_ref[...] = jnp.dot(xb, w1_ref[...], preferred_element_type=jnp.float32)
    o2_ref[...] = jnp.dot(xb, w2_ref[...], preferred_element_type=jnp.float32)
    o3_ref[...] = jnp.dot(xb, w3_ref[...], preferred_element_type=jnp.float32)


def _quad_matmul(x, ws):
    n, d = x.shape
    return pl.pallas_call(
        _quad_mm_body,
        grid=(n // BN,),
        in_specs=[pl.BlockSpec((BN, d), lambda i: (i, 0))]
        + [pl.BlockSpec((d, HW), lambda i: (0, 0))] * 4,
        out_specs=[pl.BlockSpec((BN, HW), lambda i: (i, 0))] * 4,
        out_shape=[jax.ShapeDtypeStruct((n, HW), jnp.float32)] * 4,
    )(x, *ws)


def _make_edge_kernel(n_phase):
    mesh = plsc.VectorSubcoreMesh(core_axis_name="c", subcore_axis_name="s")
    scratch = (
        [pltpu.VMEM((RPT, CH), jnp.int32)] * 2
        + [pltpu.VMEM((CH, HW), jnp.float32)] * 2
        + [pltpu.VMEM((CH, HW), jnp.float32)] * 2
        + [pltpu.VMEM((CH, W), jnp.float32)] * 2
        + [pltpu.VMEM((n_phase * V, 16), jnp.float32)]
        + [pltpu.VMEM_SHARED((NACC, W), jnp.float32)]
        + [pltpu.SemaphoreType.DMA] * 6
    )

    @functools.partial(
        pl.kernel,
        out_type=jax.ShapeDtypeStruct((n_phase * NC * NP, W), jnp.float32),
        mesh=mesh,
        scratch_types=scratch,
        compiler_params=pltpu.CompilerParams(use_tc_tiling_on_sc=False),
    )
    def edge_kernel(*refs):
        tables = refs[:2 * n_phase]
        (srow_hbm, drow_hbm, att_hbm, out_hbm) = refs[2 * n_phase:2 * n_phase + 4]
        sc = refs[2 * n_phase + 4:]
        idx_all_s = sc[0]
        idx_all_d = sc[1]
        xj_v = sc[2:4]
        xi_v = sc[4:6]
        acc_v = sc[6:8]
        att_v = sc[8]
        accum = sc[9]
        semgs = sc[10:12]
        semgd = sc[12:14]
        semsc = sc[14:16]

        c = lax.axis_index("c")
        s = lax.axis_index("s")
        wid = s * NC + c
        rbase = wid * RPT
        zero16 = jnp.zeros((16,), jnp.float32)
        lane = lax.broadcasted_iota(jnp.int32, (16,), 0)
        ind0 = jnp.maximum(1 - lane, 0).astype(jnp.float32)
        nbase = 624 * s + 8 * jnp.maximum(s - 13, 0)

        pltpu.sync_copy(att_hbm, att_v)
        pltpu.sync_copy(srow_hbm.at[pl.ds(rbase, RPT)], idx_all_s)
        pltpu.sync_copy(drow_hbm.at[pl.ds(rbase, RPT)], idx_all_d)

        def _zrow(i, carry):
            for k in range(W // 16):
                acc_v[0][i, pl.ds(16 * k, 16)] = zero16
            return carry

        for ph in range(n_phase):
            xl_hbm = tables[2 * ph]
            xr_hbm = tables[2 * ph + 1]
            att_r = [att_v[V * ph + k, :] for k in range(V)]

            lax.fori_loop(0, CH, _zrow, 0)
            for t in range(4):
                pltpu.sync_copy(acc_v[0], accum.at[pl.ds(nbase + t * CH, CH)])
            pltpu.sync_copy(acc_v[0].at[pl.ds(0, 112)],
                            accum.at[pl.ds(nbase + 512, 112)])

            @pl.when(s >= 13)
            def _():
                pltpu.sync_copy(acc_v[0].at[pl.ds(0, 8)],
                                accum.at[pl.ds(nbase + 624, 8)])

            plsc.subcore_barrier()

            def _issue_gather(q, b):
                pltpu.async_copy(xl_hbm.at[idx_all_s.at[q]], xj_v[b],
                                 semgs[b])
                pltpu.async_copy(xr_hbm.at[idx_all_d.at[q]], xi_v[b],
                                 semgd[b])

            def _wait_gather(b):
                pltpu.make_async_copy(xl_hbm.at[idx_all_s.at[0]], xj_v[b],
                                      semgs[b]).wait()
                pltpu.make_async_copy(xr_hbm.at[idx_all_d.at[0]], xi_v[b],
                                      semgd[b]).wait()

            def _wait_scatter(b):
                pltpu.make_async_copy(acc_v[b], accum.at[idx_all_d.at[0]],
                                      semsc[b]).wait()

            def _compute_scatter(q, b):

                def _edge(e, ecarry):
                    xj_r = [xj_v[b][e, pl.ds(16 * k, 16)] for k in range(V)]
                    xi_r = [xi_v[b][e, pl.ds(16 * k, 16)] for k in range(V)]
                    tot = zero16
                    for k in range(V):
                        sv = xi_r[k] + xj_r[k]
                        ev = jnp.maximum(sv, 0.2 * sv)
                        tot = tot + ev * att_r[k]
                    for step in (8, 4, 2, 1):
                        tot = tot + tot.at[lane ^ step].get(
                            mode="promise_in_bounds")
                    pv = jnp.exp(tot)
                    for k in range(V):
                        acc_v[b][e, pl.ds(16 * k, 16)] = xj_r[k] * pv
                    acc_v[b][e, pl.ds(HW, 16)] = pv * ind0
                    return ecarry

                lax.fori_loop(0, CH, _edge, 0, unroll=2)
                pltpu.async_copy(acc_v[b], accum.at[idx_all_d.at[q]],
                                 semsc[b], add=True)

            _issue_gather(0, 0)

            def _body(t, carry):
                q0 = 2 * t
                for i in range(2):
                    b = i

                    @pl.when(t >= 1)
                    def _():
                        _wait_scatter(b)

                    _issue_gather(q0 + i + 1, (i + 1) % 2)
                    _wait_gather(b)
                    _compute_scatter(q0 + i, b)
                return carry

            lax.fori_loop(0, (RPT - 2) // 2, _body, 0)
            for i in range(2):
                q = RPT - 2 + i
                _wait_scatter(i)
                if i == 0:
                    _issue_gather(q + 1, (i + 1) % 2)
                _wait_gather(i)
                _compute_scatter(q, i)
            for i in range(2):
                _wait_scatter(i)
            plsc.subcore_barrier()
            obase = (ph * NC + c) * NP + nbase
            for t in range(4):
                pltpu.sync_copy(accum.at[pl.ds(nbase + t * CH, CH)],
                                out_hbm.at[pl.ds(obase + t * CH, CH)])
            pltpu.sync_copy(accum.at[pl.ds(nbase + 512, 112)],
                            out_hbm.at[pl.ds(obase + 512, 112)])

            @pl.when(s >= 13)
            def _():
                pltpu.sync_copy(accum.at[pl.ds(nbase + 624, 8)],
                                out_hbm.at[pl.ds(obase + 624, 8)])

    return edge_kernel


_edge_l1 = _make_edge_kernel(HEADS)
_edge_l2 = _make_edge_kernel(1)


def _combine1_body(p00_ref, p01_ref, p10_ref, p11_ref, b_ref, wl_ref, wr_ref,
                   hl_ref, hr_ref):
    acc0 = p00_ref[...] + p01_ref[...]
    acc1 = p10_ref[...] + p11_ref[...]
    h0 = acc0[:, :HID] / acc0[:, HID:HID + 1]
    h1 = acc1[:, :HID] / acc1[:, HID:HID + 1]
    h = jnp.concatenate([h0, h1], axis=1) + b_ref[...]
    h = jnp.where(h > 0, h, 0.01 * h)
    hl_ref[...] = jnp.dot(h, wl_ref[...], preferred_element_type=jnp.float32)
    hr_ref[...] = jnp.dot(h, wr_ref[...], preferred_element_type=jnp.float32)


def _combine1(parts, b, wl, wr):
    off = NP // BC

    def spec(q):
        return pl.BlockSpec((BC, W), lambda i, q=q: (i + q * off, 0))

    return pl.pallas_call(
        _combine1_body,
        grid=(NP // BC,),
        in_specs=[
            spec(0), spec(1), spec(2), spec(3),
            pl.BlockSpec((1, HEADS * HID), lambda i: (0, 0)),
            pl.BlockSpec((HEADS * HID, D_OUT), lambda i: (0, 0)),
            pl.BlockSpec((HEADS * HID, D_OUT), lambda i: (0, 0)),
        ],
        out_specs=[pl.BlockSpec((BC, D_OUT), lambda i: (i, 0))] * 2,
        out_shape=[jax.ShapeDtypeStruct((NP, D_OUT), jnp.float32)] * 2,
    )(parts, parts, parts, parts, b, wl, wr)


def _combine2_body(p0_ref, p1_ref, b_ref, o_ref):
    acc = p0_ref[...] + p1_ref[...]
    o_ref[...] = acc[:, :D_OUT] / acc[:, D_OUT:D_OUT + 1] + b_ref[...]


def _combine2(parts, b):
    off = NP // BC
    return pl.pallas_call(
        _combine2_body,
        grid=(N // BC,),
        in_specs=[
            pl.BlockSpec((BC, W), lambda i: (i, 0)),
            pl.BlockSpec((BC, W), lambda i: (i + off, 0)),
            pl.BlockSpec((1, D_OUT), lambda i: (0, 0)),
        ],
        out_specs=pl.BlockSpec((BC, D_OUT), lambda i: (i, 0)),
        out_shape=jax.ShapeDtypeStruct((N, D_OUT), jnp.float32),
    )(parts, parts, b)


def kernel(x, edge_index, Wl1, Wr1, att1, b1, Wl2, Wr2, att2, b2):
    xf = jnp.pad(x.reshape(N, D_IN), ((0, NP - N), (0, 0)))
    loop = jnp.arange(N, dtype=edge_index.dtype)
    src = jnp.concatenate([edge_index[0], loop])
    dst = jnp.concatenate([edge_index[1], loop])
    pad = (R + 2) * CH - ET
    src = jnp.pad(src, (0, pad), constant_values=N).reshape(R + 2, CH)
    dst = jnp.pad(dst, (0, pad), constant_values=N).reshape(R + 2, CH)

    xl0, xl1, xr0, xr1 = _quad_matmul(
        xf, (Wl1[:, :HID], Wl1[:, HID:], Wr1[:, :HID], Wr1[:, HID:]))
    parts1 = _edge_l1(xl0, xr0, xl1, xr1, src, dst,
                      att1.reshape(HEADS * V, 16))
    hl2, hr2 = _combine1(parts1, b1.reshape(1, -1), Wl2, Wr2)
    parts2 = _edge_l2(hl2, hr2, src, dst, att2.reshape(V, 16))
    out = _combine2(parts2, b2.reshape(1, -1))
    return out.reshape(1, N, D_OUT)
